# baseline (speedup 1.0000x reference)
import functools

import jax
import jax.numpy as jnp
from jax import lax
from jax.experimental import pallas as pl
from jax.experimental.pallas import tpu as pltpu
from jax.experimental.pallas import tpu_sc as plsc


def _pattern_body(vx_ref, g_ref, vy_ref, out_ref):
    H = vx_ref.shape[0]
    W = vy_ref.shape[0]
    vx = vx_ref[...]
    gx = jnp.stack([g_ref[i, 0] for i in range(H)])[:, None]
    gy = jnp.stack([g_ref[i, 1] for i in range(W)])[:, None]
    wx = gx * vx * jax.lax.rsqrt(jnp.sum(vx * vx, axis=1, keepdims=True))
    vy = vy_ref[...]
    wy = gy * vy * jax.lax.rsqrt(jnp.sum(vy * vy, axis=1, keepdims=True))
    xblock = jnp.tile(wx, (W, 1))
    yblock = jnp.repeat(wy, H, axis=0)
    out_ref[...] = jnp.concatenate([xblock, yblock], axis=1)


def kernel(inp, vx, gx, vy, gy):
    b = inp.shape[0]
    H, D = vx.shape
    W = vy.shape[0]
    rows, width = W * H, 2 * D
    g2 = jnp.concatenate([gx, gy], axis=1)

    full = lambda s: pl.BlockSpec(s, lambda: (0,) * len(s))
    pattern = pl.pallas_call(
        _pattern_body,
        in_specs=[full((H, D)),
                  pl.BlockSpec(memory_space=pltpu.MemorySpace.SMEM),
                  full((W, D))],
        out_specs=full((rows, width)),
        out_shape=jax.ShapeDtypeStruct((rows, width), jnp.float32),
    )(vx, g2, vy)

    info = plsc.get_sparse_core_info()
    NW = info.num_cores * info.num_subcores
    rpw = rows // NW

    @functools.partial(
        pl.kernel,
        mesh=plsc.VectorSubcoreMesh(core_axis_name="c", subcore_axis_name="s"),
        out_type=jax.ShapeDtypeStruct((b, rows, width), jnp.float32),
        scratch_types=[
            pltpu.VMEM((rpw, width), jnp.float32),
            pltpu.SemaphoreType.DMA,
        ],
    )
    def sc_broadcast(pattern_hbm, out_hbm, chunk, sem):
        wid = lax.axis_index("s") * info.num_cores + lax.axis_index("c")
        base = wid * rpw
        pltpu.sync_copy(pattern_hbm.at[pl.ds(base, rpw)], chunk)
        descs = [
            pltpu.async_copy(chunk, out_hbm.at[i, pl.ds(base, rpw)], sem)
            for i in range(b)
        ]
        for d in descs:
            d.wait()

    return sc_broadcast(pattern)


# speedup vs baseline: 1.0013x; 1.0013x over previous
import functools

import jax
import jax.numpy as jnp
from jax import lax
from jax.experimental import pallas as pl
from jax.experimental.pallas import tpu as pltpu
from jax.experimental.pallas import tpu_sc as plsc


def _pattern_body(vx_ref, g_ref, vy_ref, out_ref):
    H = vx_ref.shape[0]
    W = vy_ref.shape[0]
    vx = vx_ref[...]
    gx = g_ref[:, 0:1]
    gy = g_ref[:, 1:2]
    wx = gx * vx * jax.lax.rsqrt(jnp.sum(vx * vx, axis=1, keepdims=True))
    vy = vy_ref[...]
    wy = gy * vy * jax.lax.rsqrt(jnp.sum(vy * vy, axis=1, keepdims=True))
    xblock = jnp.tile(wx, (W, 1))
    yblock = jnp.repeat(wy, H, axis=0)
    out_ref[...] = jnp.concatenate([xblock, yblock], axis=1)


def kernel(inp, vx, gx, vy, gy):
    b = inp.shape[0]
    H, D = vx.shape
    W = vy.shape[0]
    rows, width = W * H, 2 * D
    g2 = jnp.concatenate([gx, gy], axis=1)

    full = lambda s: pl.BlockSpec(s, lambda: (0,) * len(s))
    pattern = pl.pallas_call(
        _pattern_body,
        in_specs=[full((H, D)), full((H, 2)), full((W, D))],
        out_specs=full((rows, width)),
        out_shape=jax.ShapeDtypeStruct((rows, width), jnp.float32),
    )(vx, g2, vy)

    info = plsc.get_sparse_core_info()
    NW = info.num_cores * info.num_subcores
    rpw = rows // NW

    @functools.partial(
        pl.kernel,
        mesh=plsc.VectorSubcoreMesh(core_axis_name="c", subcore_axis_name="s"),
        out_type=jax.ShapeDtypeStruct((b, rows, width), jnp.float32),
        scratch_types=[
            pltpu.VMEM((rpw, width), jnp.float32),
            pltpu.SemaphoreType.DMA,
        ],
    )
    def sc_broadcast(pattern_hbm, out_hbm, chunk, sem):
        wid = lax.axis_index("s") * info.num_cores + lax.axis_index("c")
        base = wid * rpw
        pltpu.sync_copy(pattern_hbm.at[pl.ds(base, rpw)], chunk)
        descs = [
            pltpu.async_copy(chunk, out_hbm.at[i, pl.ds(base, rpw)], sem)
            for i in range(b)
        ]
        for d in descs:
            d.wait()

    return sc_broadcast(pattern)
